# Initial kernel scaffold; baseline (speedup 1.0000x reference)
import jax
import jax.numpy as jnp
from jax.experimental import pallas as pl


def kernel(gene_ids, expression_bins, gene_table, expr_table, pos_table, ln_gamma, ln_beta):
    raise NotImplementedError("write your pallas kernel here")


# speedup vs baseline: 2.7802x; 2.7802x over previous
import functools

import jax
import jax.numpy as jnp
from jax import lax
from jax.experimental import pallas as pl
from jax.experimental.pallas import tpu as pltpu
from jax.experimental.pallas import tpu_sc as plsc

_NC = 2
_NS = 16
_NW = _NC * _NS
_L = 16

_EPS = 1e-12


def _rsqrt16(v):
    i = plsc.bitcast(v, jnp.int32)
    i = jnp.int32(0x5F3759DF) - (i >> 1)
    y = plsc.bitcast(i, jnp.float32)
    for _ in range(3):
        y = y * (1.5 - 0.5 * v * y * y)
    return y


def _build_sc_call(B, S, H, VOCAB, NBINS):
    rows_per_w = B // _NW
    n_chunks = 2
    chunk = S // n_chunks
    mesh = plsc.VectorSubcoreMesh(
        core_axis_name="c", subcore_axis_name="s",
        num_cores=_NC, num_subcores=_NS)

    @functools.partial(
        pl.kernel,
        out_type=jax.ShapeDtypeStruct((B, S, H), jnp.float32),
        mesh=mesh,
        compiler_params=pltpu.CompilerParams(needs_layout_passes=False),
        scratch_types=[
            pltpu.VMEM((n_chunks, chunk), jnp.int32),
            pltpu.VMEM((S,), jnp.int32),
            pltpu.VMEM((S, H), jnp.float32),
            pltpu.VMEM((NBINS, H), jnp.float32),
            pltpu.VMEM((S, H), jnp.float32),
            pltpu.VMEM((2, H), jnp.float32),
            pltpu.SemaphoreType.DMA,
        ],
    )
    def sc_kernel(ids_hbm, bins_hbm, gene_hbm, expr_hbm, pos_hbm, gam_hbm,
                  bet_hbm, out_hbm, idx_v, bins_v, grows, expr_v, pos_v,
                  gb_v, gsem):
        wid = lax.axis_index("s") * _NC + lax.axis_index("c")

        pltpu.sync_copy(expr_hbm, expr_v)
        pltpu.sync_copy(pos_hbm.at[pl.ds(0, S)], pos_v)
        pltpu.sync_copy(gam_hbm, gb_v.at[0])
        pltpu.sync_copy(bet_hbm, gb_v.at[1])

        off16 = lax.iota(jnp.int32, _L)
        gams = [gb_v[0, pl.ds(16 * j, 16)] for j in range(H // _L)]
        bets = [gb_v[1, pl.ds(16 * j, 16)] for j in range(H // _L)]

        def row_body(r, carry):
            row = wid * rows_per_w + r
            pltpu.sync_copy(ids_hbm.at[row], idx_v)
            pltpu.sync_copy(bins_hbm.at[row], bins_v)
            cps = [
                pltpu.async_copy(
                    gene_hbm.at[idx_v.at[k]],
                    grows.at[pl.ds(k * chunk, chunk)], gsem)
                for k in range(n_chunks)
            ]
            for cp in cps:
                cp.wait()

            def tok_body(s, c):
                ssplat = jnp.full((_L,), s, dtype=jnp.int32)
                binv = plsc.load_gather(bins_v, [ssplat])
                xs = []
                s1 = jnp.zeros((_L,), jnp.float32)
                s2 = jnp.zeros((_L,), jnp.float32)
                for j in range(H // _L):
                    offj = off16 + jnp.int32(16 * j)
                    ev = plsc.load_gather(expr_v, [binv, offj])
                    gv = grows[s, pl.ds(16 * j, 16)]
                    pv = pos_v[s, pl.ds(16 * j, 16)]
                    x = gv + ev + pv
                    xs.append(x)
                    s1 = s1 + x
                    s2 = s2 + x * x
                hs1 = jnp.sum(s1)
                hs2 = jnp.sum(s2)
                mean = hs1 * jnp.float32(1.0 / H)
                var = hs2 * jnp.float32(1.0 / H) - mean * mean
                inv = _rsqrt16(jnp.full((_L,), var + jnp.float32(_EPS),
                                        dtype=jnp.float32))
                meanv = jnp.full((_L,), mean, dtype=jnp.float32)
                for j in range(H // _L):
                    y = (xs[j] - meanv) * inv
                    grows[s, pl.ds(16 * j, 16)] = y * gams[j] + bets[j]
                return c

            lax.fori_loop(0, S, tok_body, 0)
            pltpu.sync_copy(grows, out_hbm.at[row])
            return carry

        lax.fori_loop(0, rows_per_w, row_body, 0)

    return sc_kernel


def kernel(gene_ids, expression_bins, gene_table, expr_table, pos_table,
           ln_gamma, ln_beta):
    B, S = gene_ids.shape
    VOCAB, H = gene_table.shape
    NBINS = expr_table.shape[0]
    ids2 = gene_ids.reshape(B, 2, S // 2)
    fn = _build_sc_call(B, S, H, VOCAB, NBINS)
    return fn(ids2, expression_bins, gene_table, expr_table, pos_table,
              ln_gamma, ln_beta)


# speedup vs baseline: 3.7224x; 1.3389x over previous
import functools

import jax
import jax.numpy as jnp
import numpy as np
from jax import lax
from jax.experimental import pallas as pl
from jax.experimental.pallas import tpu as pltpu
from jax.experimental.pallas import tpu_sc as plsc

_NC = 2
_NS = 16
_NW = _NC * _NS
_L = 16

_EPS = 1e-12


def _rsqrt16(v):
    i = plsc.bitcast(v, jnp.int32)
    i = jnp.int32(0x5F3759DF) - (i >> 1)
    y = plsc.bitcast(i, jnp.float32)
    for _ in range(2):
        y = y * (1.5 - 0.5 * v * y * y)
    return y


def _build_sc_call(B, S, H, VOCAB, NBINS):
    rows_per_w = B // _NW
    n_chunks = 2
    chunk = S // n_chunks
    nj = H // _L
    mesh = plsc.VectorSubcoreMesh(
        core_axis_name="c", subcore_axis_name="s",
        num_cores=_NC, num_subcores=_NS)

    @functools.partial(
        pl.kernel,
        out_type=jax.ShapeDtypeStruct((B, S, H), jnp.float32),
        mesh=mesh,
        compiler_params=pltpu.CompilerParams(needs_layout_passes=False),
        scratch_types=[
            pltpu.VMEM((rows_per_w, n_chunks, chunk), jnp.int32),
            pltpu.VMEM((rows_per_w, S), jnp.int32),
            pltpu.VMEM((S, H), jnp.float32),
            pltpu.VMEM((S, H), jnp.float32),
            pltpu.VMEM((S, H), jnp.float32),
            pltpu.VMEM((NBINS, H), jnp.float32),
            pltpu.VMEM((S, H), jnp.float32),
            pltpu.VMEM((2, H), jnp.float32),
            pltpu.SemaphoreType.DMA,
            pltpu.SemaphoreType.DMA,
            pltpu.SemaphoreType.DMA,
            pltpu.SemaphoreType.DMA,
            pltpu.SemaphoreType.DMA,
            pltpu.SemaphoreType.DMA,
        ],
    )
    def sc_kernel(ids_hbm, bins_hbm, gene_hbm, expr_hbm, pos_hbm, gam_hbm,
                  bet_hbm, out_hbm, ids_v, bins_v, buf0, buf1, buf2,
                  expr_v, pos_v, gb_v, g0, g1, g2, o0, o1, o2):
        wid = lax.axis_index("s") * _NC + lax.axis_index("c")
        base = wid * rows_per_w
        bufs = [buf0, buf1, buf2]
        gsems = [g0, g1, g2]
        osems = [o0, o1, o2]

        pltpu.sync_copy(ids_hbm.at[pl.ds(base, rows_per_w)], ids_v)
        pltpu.sync_copy(bins_hbm.at[pl.ds(base, rows_per_w)], bins_v)
        pltpu.sync_copy(expr_hbm, expr_v)
        pltpu.sync_copy(pos_hbm.at[pl.ds(0, S)], pos_v)
        pltpu.sync_copy(gam_hbm, gb_v.at[0])
        pltpu.sync_copy(bet_hbm, gb_v.at[1])

        off16 = lax.iota(jnp.int32, _L)
        offs = [off16 + jnp.int32(16 * j) for j in range(nj)]
        gams = [gb_v[0, pl.ds(16 * j, 16)] for j in range(nj)]
        bets = [gb_v[1, pl.ds(16 * j, 16)] for j in range(nj)]
        invh = jnp.float32(1.0 / H)

        def start_gather(b, rloc):
            for k in range(n_chunks):
                pltpu.async_copy(
                    gene_hbm.at[ids_v.at[rloc, k]],
                    bufs[b].at[pl.ds(k * chunk, chunk)], gsems[b])

        def wait_gather(b, rloc):
            for k in range(n_chunks):
                pltpu.make_async_copy(
                    gene_hbm.at[ids_v.at[rloc, k]],
                    bufs[b].at[pl.ds(k * chunk, chunk)], gsems[b]).wait()

        def start_out(b, rloc):
            pltpu.async_copy(bufs[b], out_hbm.at[base + rloc], osems[b])

        def wait_out(b, rloc):
            pltpu.make_async_copy(
                bufs[b], out_hbm.at[base + rloc], osems[b]).wait()

        def token(buf, rsplat, s):
            ssplat = jnp.full((_L,), s, dtype=jnp.int32)
            binv = plsc.load_gather(bins_v, [rsplat, ssplat])
            xs = []
            s1 = None
            s2 = None
            for j in range(nj):
                ev = plsc.load_gather(expr_v, [binv, offs[j]])
                gv = buf[s, pl.ds(16 * j, 16)]
                pv = pos_v[s, pl.ds(16 * j, 16)]
                x = gv + ev + pv
                xs.append(x)
                s1 = x if s1 is None else s1 + x
                s2 = x * x if s2 is None else s2 + x * x
            hs1 = jnp.sum(s1)
            hs2 = jnp.sum(s2)
            mean_s = hs1 * invh
            var_s = hs2 * invh - mean_s * mean_s + jnp.float32(_EPS)
            inv = _rsqrt16(jnp.full((_L,), var_s, dtype=jnp.float32))
            mean = jnp.full((_L,), mean_s, dtype=jnp.float32)
            for j in range(nj):
                y = (xs[j] - mean) * inv
                buf[s, pl.ds(16 * j, 16)] = y * gams[j] + bets[j]

        def compute(b, rloc):
            buf = bufs[b]
            rsplat = jnp.full((_L,), rloc, dtype=jnp.int32)

            def tok2(i, c):
                token(buf, rsplat, 2 * i)
                token(buf, rsplat, 2 * i + 1)
                return c

            lax.fori_loop(0, S // 2, tok2, 0, unroll=False)

        def phase(rloc, b, do_out_wait, do_gather):
            wait_gather(b, rloc)
            if do_out_wait:
                wait_out((b + 1) % 3, rloc - 2)
            if do_gather:
                start_gather((b + 1) % 3, rloc + 1)
            compute(b, rloc)
            start_out(b, rloc)

        start_gather(0, 0)
        phase(0, 0, False, True)
        phase(1, 1, False, True)

        def pipe3(r3, c):
            r = 2 + 3 * r3
            phase(r, 2, True, True)
            phase(r + 1, 0, True, True)
            phase(r + 2, 1, True, True)
            return c

        n_groups = (rows_per_w - 5) // 3
        lax.fori_loop(0, n_groups, pipe3, 0, unroll=False)
        r_tail = 2 + 3 * n_groups
        phase(r_tail, r_tail % 3, True, True)
        phase(r_tail + 1, (r_tail + 1) % 3, True, True)
        phase(r_tail + 2, (r_tail + 2) % 3, False, False)
        wait_out(r_tail % 3, r_tail)
        wait_out((r_tail + 1) % 3, r_tail + 1)
        wait_out((r_tail + 2) % 3, r_tail + 2)

    return sc_kernel


def kernel(gene_ids, expression_bins, gene_table, expr_table, pos_table,
           ln_gamma, ln_beta):
    B, S = gene_ids.shape
    VOCAB, H = gene_table.shape
    NBINS = expr_table.shape[0]
    ids2 = gene_ids.reshape(B, 2, S // 2)
    fn = _build_sc_call(B, S, H, VOCAB, NBINS)
    return fn(ids2, expression_bins, gene_table, expr_table, pos_table,
              ln_gamma, ln_beta)
